# baseline (speedup 1.0000x reference)
import functools

import jax
import jax.numpy as jnp
from jax.experimental import pallas as pl
from jax.experimental.pallas import tpu as pltpu

BN_EPS = 1e-5


def _affine_from_stats(sp_ref, ssp_ref, g_ref, be_ref, cnt, total):
    C = sp_ref.shape[-1]
    s = sp_ref[:, 0, :]
    ss = ssp_ref[:, 0, :]
    mean_p = s * (1.0 / cnt)
    m2_p = ss - s * mean_p
    mean = jnp.sum(s, axis=0, keepdims=True) * (1.0 / total)
    m2 = (jnp.sum(m2_p, axis=0, keepdims=True)
          + cnt * jnp.sum((mean_p - mean) ** 2, axis=0, keepdims=True))
    var = m2 * (1.0 / total)
    scale = g_ref[...].reshape(1, C) * jax.lax.rsqrt(var + BN_EPS)
    shift = be_ref[...].reshape(1, C) - mean * scale
    return scale, shift


def _conv_stage_kernel(xb_ref, sp_ref, ssp_ref, g_ref, be_ref, w_ref, b_ref,
                       y_ref, s_ref, ss_ref, scr_ref,
                       *, act_input, tr, cnt, total):
    nb, H, W, Ci = xb_ref.shape
    Co = w_ref.shape[-1] // 3
    M = tr * W
    iota = jax.lax.broadcasted_iota(jnp.int32, (M, 1), 0)
    mask_l = (iota % W != 0).astype(jnp.float32)
    mask_r = (iota % W != W - 1).astype(jnp.float32)
    bias = b_ref[...]
    if act_input:
        sc_a, sh_a = _affine_from_stats(sp_ref, ssp_ref, g_ref, be_ref,
                                        cnt, total)

    for img in range(nb):
        xb = xb_ref[img]
        if act_input:
            sc = sc_a.reshape(1, 1, Ci)
            sh = sh_a.reshape(1, 1, Ci)
            xb = jnp.maximum(xb.astype(jnp.float32) * sc + sh, 0.0)
        xb = xb.astype(jnp.bfloat16)
        scr_ref[:, :, Ci:2 * Ci] = xb
        scr_ref[1:H, :, 0:Ci] = xb[0:H - 1]
        scr_ref[0:1, :, 0:Ci] = jnp.zeros((1, W, Ci), jnp.bfloat16)
        scr_ref[0:H - 1, :, 2 * Ci:3 * Ci] = xb[1:H]
        scr_ref[H - 1:H, :, 2 * Ci:3 * Ci] = jnp.zeros((1, W, Ci), jnp.bfloat16)

        s_tot = jnp.zeros((1, Co), jnp.float32)
        ss_tot = jnp.zeros((1, Co), jnp.float32)
        for r0 in range(0, H, tr):
            lhs = scr_ref[r0:r0 + tr].reshape(M, 3 * Ci)
            pair = jnp.dot(lhs, w_ref[:, 0:2 * Co],
                           preferred_element_type=jnp.float32)
            acc = jnp.dot(lhs, w_ref[:, 2 * Co:3 * Co],
                          preferred_element_type=jnp.float32)
            acc = acc + jnp.roll(pair[:, 0:Co], 1, axis=0) * mask_l
            acc = acc + jnp.roll(pair[:, Co:2 * Co], -1, axis=0) * mask_r
            acc = acc + bias
            y_ref[img, r0:r0 + tr, :, :] = (
                acc.reshape(tr, W, Co).astype(jnp.bfloat16))
            s_tot = s_tot + jnp.sum(acc, axis=0, keepdims=True)
            ss_tot = ss_tot + jnp.sum(acc * acc, axis=0, keepdims=True)

        s_ref[img] = jnp.broadcast_to(s_tot.reshape(1, Co), (8, Co))
        ss_ref[img] = jnp.broadcast_to(ss_tot.reshape(1, Co), (8, Co))


def _conv_stage(x, s_prev, ss_prev, g, be, w_packed, b, *, act_input, tr):
    N, H, W, Ci = x.shape
    Co = w_packed.shape[-1] // 3
    cnt = float(H * W)
    total = float(N * H * W)

    nb = 4 if N % 4 == 0 else 1
    body = functools.partial(_conv_stage_kernel, act_input=act_input, tr=tr,
                             cnt=cnt, total=total)
    return pl.pallas_call(
        body,
        grid=(N // nb,),
        in_specs=[
            pl.BlockSpec((nb, H, W, Ci), lambda n: (n, 0, 0, 0)),
            pl.BlockSpec(s_prev.shape, lambda n: (0, 0, 0)),
            pl.BlockSpec(ss_prev.shape, lambda n: (0, 0, 0)),
            pl.BlockSpec((1, Ci), lambda n: (0, 0)),
            pl.BlockSpec((1, Ci), lambda n: (0, 0)),
            pl.BlockSpec((3 * Ci, 3 * Co), lambda n: (0, 0)),
            pl.BlockSpec((1, Co), lambda n: (0, 0)),
        ],
        out_specs=(
            pl.BlockSpec((nb, H, W, Co), lambda n: (n, 0, 0, 0)),
            pl.BlockSpec((nb, 8, Co), lambda n: (n, 0, 0)),
            pl.BlockSpec((nb, 8, Co), lambda n: (n, 0, 0)),
        ),
        out_shape=(
            jax.ShapeDtypeStruct((N, H, W, Co), jnp.bfloat16),
            jax.ShapeDtypeStruct((N, 8, Co), jnp.float32),
            jax.ShapeDtypeStruct((N, 8, Co), jnp.float32),
        ),
        scratch_shapes=[
            pltpu.VMEM((H, W, 3 * Ci), jnp.bfloat16),
        ],
        compiler_params=pltpu.CompilerParams(
            dimension_semantics=("parallel",),
            vmem_limit_bytes=48 * 1024 * 1024),
    )(x, s_prev, ss_prev, g, be, w_packed, b)


def _norm_relu_kernel(y_ref, sp_ref, ssp_ref, g_ref, be_ref, o_ref,
                      *, cnt, total):
    C = y_ref.shape[-1]
    sc, sh = _affine_from_stats(sp_ref, ssp_ref, g_ref, be_ref, cnt, total)
    v = (y_ref[...].astype(jnp.float32) * sc.reshape(1, 1, 1, C)
         + sh.reshape(1, 1, 1, C))
    o_ref[...] = jnp.maximum(v, 0.0).astype(jnp.bfloat16)


def _norm_relu(y, s_prev, ss_prev, g, be):
    N, H, W, C = y.shape
    cnt = float(H * W)
    total = float(N * H * W)
    nb = 4 if N % 4 == 0 else 1
    body = functools.partial(_norm_relu_kernel, cnt=cnt, total=total)
    return pl.pallas_call(
        body,
        grid=(N // nb,),
        in_specs=[
            pl.BlockSpec((nb, H, W, C), lambda n: (n, 0, 0, 0)),
            pl.BlockSpec(s_prev.shape, lambda n: (0, 0, 0)),
            pl.BlockSpec(ss_prev.shape, lambda n: (0, 0, 0)),
            pl.BlockSpec((1, C), lambda n: (0, 0)),
            pl.BlockSpec((1, C), lambda n: (0, 0)),
        ],
        out_specs=pl.BlockSpec((nb, H, W, C), lambda n: (n, 0, 0, 0)),
        out_shape=jax.ShapeDtypeStruct((N, H, W, C), jnp.bfloat16),
        compiler_params=pltpu.CompilerParams(
            dimension_semantics=("parallel",),
            vmem_limit_bytes=32 * 1024 * 1024),
    )(y, s_prev, ss_prev, g, be)


def _pack_w(w):
    slabs = [jnp.concatenate([w[dy, dx] for dy in range(3)], axis=0)
             for dx in range(3)]
    return jnp.concatenate([slabs[0], slabs[2], slabs[1]],
                           axis=1).astype(jnp.bfloat16)


def kernel(x, w1, b1, g1, be1, w2, b2, g2, be2):
    N, Cin, H, W = x.shape
    Cout = w1.shape[-1]
    tr = 4 if (H % 4 == 0) else 1

    xh = jnp.transpose(x, (0, 2, 3, 1)).astype(jnp.bfloat16)

    w1p = _pack_w(w1)
    w2p = _pack_w(w2)
    b1r = b1.reshape(1, Cout).astype(jnp.float32)
    b2r = b2.reshape(1, Cout).astype(jnp.float32)
    g1r = g1.reshape(1, Cout).astype(jnp.float32)
    be1r = be1.reshape(1, Cout).astype(jnp.float32)
    g2r = g2.reshape(1, Cout).astype(jnp.float32)
    be2r = be2.reshape(1, Cout).astype(jnp.float32)
    zs = jnp.zeros((1, 8, Cin), jnp.float32)
    za = jnp.zeros((1, Cin), jnp.float32)

    y1, s1, ss1 = _conv_stage(xh, zs, zs, za, za, w1p, b1r,
                              act_input=False, tr=tr)
    y2, s2, ss2 = _conv_stage(y1, s1, ss1, g1r, be1r, w2p, b2r,
                              act_input=True, tr=tr)
    out = _norm_relu(y2, s2, ss2, g2r, be2r)
    return jnp.transpose(out, (0, 3, 1, 2)).astype(jnp.float32)
